# baseline (speedup 1.0000x reference)
import functools

import jax
import jax.numpy as jnp
from jax import lax
from jax.experimental import pallas as pl
from jax.experimental.pallas import tpu as pltpu
from jax.experimental.pallas import tpu_sc as plsc

NC = 2
NS = 16
NW = NC * NS


def _tc1_body(x_ref, w0_ref, b0_ref, h_ref):
    h_ref[...] = jax.nn.relu(
        jnp.dot(x_ref[...], w0_ref[...], preferred_element_type=jnp.float32,
                precision=lax.Precision.HIGHEST)
        + b0_ref[...]
    )


def _tc2_body(h_ref, p_ref, wroot_ref, we_ref, bconv_ref, out_ref):
    a = p_ref[0] + p_ref[1]
    out_ref[...] = (
        jnp.dot(h_ref[...], wroot_ref[...], preferred_element_type=jnp.float32,
                precision=lax.Precision.HIGHEST)
        + jnp.dot(a, we_ref[...], preferred_element_type=jnp.float32,
                precision=lax.Precision.HIGHEST)
        + bconv_ref[...]
    )


def _tc3_body(emb_ref, w1_ref, b1_ref, w2_ref, b2_ref, out_ref):
    ee = jax.nn.relu(
        jnp.dot(emb_ref[...], w1_ref[...], preferred_element_type=jnp.float32,
                precision=lax.Precision.HIGHEST)
        + b1_ref[...]
    )
    out_ref[...] = (
        jnp.dot(ee, w2_ref[...], preferred_element_type=jnp.float32,
                precision=lax.Precision.HIGHEST)
        + b2_ref[...]
    )


CHUNK = 128


def _sc_segsum_body(h_hbm, src2_hbm, dst2_hbm, part_hbm,
                    sidx_v, didx_v, rows0, rows1, zrow_v, acc_sh,
                    g0, g1, s0, s1,
                    *, nch, rows_per_sub, rows_last):
    cid = lax.axis_index("c")
    sid = lax.axis_index("s")
    wid = sid * NC + cid

    zchunk = zrow_v.shape[0]

    @pl.loop(0, zchunk)
    def _zero_buf(i):
        zrow_v[i, :] = jnp.zeros((16,), jnp.float32)

    @pl.loop(0, rows_per_sub // zchunk)
    def _zero_acc(k):
        pltpu.sync_copy(zrow_v, acc_sh.at[pl.ds(sid * rows_per_sub + k * zchunk, zchunk)])

    base = wid * nch
    pltpu.sync_copy(src2_hbm.at[pl.ds(base, nch)], sidx_v)
    pltpu.sync_copy(dst2_hbm.at[pl.ds(base, nch)], didx_v)
    plsc.subcore_barrier()

    rows = (rows0, rows1)
    gsem = (g0, g1)
    ssem = (s0, s1)
    pltpu.async_copy(h_hbm.at[sidx_v.at[0]], rows0, g0)

    @pl.loop(0, nch - 1, step=2)
    def _main(jj):
        for b in range(2):
            j = jj + b
            cur, oth = rows[b], rows[1 - b]
            pltpu.make_async_copy(h_hbm.at[sidx_v.at[j]], cur, gsem[b]).wait()
            pltpu.async_copy(cur, acc_sh.at[didx_v.at[j]], ssem[b], add=True)

            @pl.when(j >= 1)
            def _w():
                pltpu.make_async_copy(oth, acc_sh.at[didx_v.at[j]],
                                      ssem[1 - b]).wait()

            pltpu.async_copy(h_hbm.at[sidx_v.at[j + 1]], oth, gsem[1 - b])

    lb = (nch - 1) % 2
    cur, oth = rows[lb], rows[1 - lb]
    pltpu.make_async_copy(h_hbm.at[sidx_v.at[nch - 1]], cur, gsem[lb]).wait()
    pltpu.async_copy(cur, acc_sh.at[didx_v.at[nch - 1]], ssem[lb], add=True)
    pltpu.make_async_copy(oth, acc_sh.at[didx_v.at[nch - 1]], ssem[1 - lb]).wait()
    pltpu.make_async_copy(cur, acc_sh.at[didx_v.at[nch - 1]], ssem[lb]).wait()

    plsc.subcore_barrier()

    @pl.when(sid < NS - 1)
    def _full():
        pltpu.sync_copy(acc_sh.at[pl.ds(sid * rows_per_sub, rows_per_sub)],
                        part_hbm.at[cid, pl.ds(sid * rows_per_sub, rows_per_sub)])

    @pl.when(sid == NS - 1)
    def _clipped():
        pltpu.sync_copy(acc_sh.at[pl.ds((NS - 1) * rows_per_sub, rows_last)],
                        part_hbm.at[cid, pl.ds((NS - 1) * rows_per_sub, rows_last)])


def _sc_edgemul_body(out_hbm, src2_hbm, dst2_hbm, emb_hbm,
                     sidx_v, didx_v, sr0, sr1, dr0, dr1,
                     gs0, gs1, gd0, gd1, w0, w1,
                     *, nch):
    cid = lax.axis_index("c")
    sid = lax.axis_index("s")
    wid = sid * NC + cid

    base = wid * nch
    pltpu.sync_copy(src2_hbm.at[pl.ds(base, nch)], sidx_v)
    pltpu.sync_copy(dst2_hbm.at[pl.ds(base, nch)], didx_v)

    srows = (sr0, sr1)
    drows = (dr0, dr1)
    gsS = (gs0, gs1)
    gsD = (gd0, gd1)
    wsem = (w0, w1)

    def _wr_dst(j):
        return emb_hbm.at[pl.ds((base + j) * CHUNK, CHUNK)]

    pltpu.async_copy(out_hbm.at[sidx_v.at[0]], sr0, gs0)
    pltpu.async_copy(out_hbm.at[didx_v.at[0]], dr0, gd0)

    def _process(j, b):
        cur_s, cur_d = srows[b], drows[b]
        pltpu.make_async_copy(out_hbm.at[sidx_v.at[j]], cur_s, gsS[b]).wait()
        pltpu.make_async_copy(out_hbm.at[didx_v.at[j]], cur_d, gsD[b]).wait()

        @pl.loop(0, CHUNK, unroll=8)
        def _mul(r):
            cur_s[r, :] = cur_s[r, :] * cur_d[r, :]

        pltpu.async_copy(cur_s, _wr_dst(j), wsem[b])

    @pl.loop(0, nch - 1, step=2)
    def _main(jj):
        for b in range(2):
            j = jj + b
            _process(j, b)

            @pl.when(j >= 1)
            def _w():
                pltpu.make_async_copy(srows[1 - b], _wr_dst(j), wsem[1 - b]).wait()

            pltpu.async_copy(out_hbm.at[sidx_v.at[j + 1]], srows[1 - b], gsS[1 - b])
            pltpu.async_copy(out_hbm.at[didx_v.at[j + 1]], drows[1 - b], gsD[1 - b])

    lb = (nch - 1) % 2
    _process(nch - 1, lb)
    pltpu.make_async_copy(srows[1 - lb], _wr_dst(nch - 1), wsem[1 - lb]).wait()
    pltpu.make_async_copy(srows[lb], _wr_dst(nch - 1), wsem[lb]).wait()


def kernel(x, edge_index, W0, b0, We1, be1, We2, be2, Wroot, bconv, W1, b1,
           W2, b2):
    n, d = x.shape
    e = edge_index.shape[1]
    h_dim = W0.shape[1]

    src = edge_index[0]
    dst = edge_index[1]

    e1 = jax.nn.relu(We1[0] + be1)
    w_e = (e1 @ We2 + be2).reshape(h_dim, h_dim)

    P = 128 // h_dim
    eyeP = jnp.eye(P, dtype=jnp.float32)
    W0big = jnp.kron(eyeP, W0)
    b0big = jnp.tile(b0, P).reshape(1, P * h_dim)
    Wrootbig = jnp.kron(eyeP, Wroot)
    Webig = jnp.kron(eyeP, w_e)
    bconvbig = jnp.tile(bconv, P).reshape(1, P * h_dim)
    W1big = jnp.kron(eyeP, W1)
    b1big = jnp.tile(b1, P).reshape(1, P * 8)
    W2big = jnp.kron(eyeP, W2)

    h_p = pl.pallas_call(
        _tc1_body,
        out_shape=jax.ShapeDtypeStruct((n // P, P * h_dim), jnp.float32),
    )(x.reshape(n // P, P * d), W0big, b0big)
    h = h_p.reshape(n, h_dim)

    chunks = -(-e // (CHUNK * NW)) * NW
    nch = chunks // NW
    pad_e = chunks * CHUNK - e
    src2 = jnp.concatenate(
        [src, jnp.zeros((pad_e,), jnp.int32)]).reshape(chunks, CHUNK)
    dstA = jnp.concatenate(
        [dst, jnp.full((pad_e,), n, jnp.int32)]).reshape(chunks, CHUNK)
    dstB = jnp.concatenate(
        [dst, jnp.zeros((pad_e,), jnp.int32)]).reshape(chunks, CHUNK)
    rows_per_sub = -(-n // (8 * NS)) * 8
    npad = rows_per_sub * NS
    zchunk = rows_per_sub // 4
    mesh = plsc.VectorSubcoreMesh(core_axis_name="c", subcore_axis_name="s",
                                  num_cores=NC, num_subcores=NS)
    rows_last = n - rows_per_sub * (NS - 1)
    seg = functools.partial(_sc_segsum_body, nch=nch,
                            rows_per_sub=rows_per_sub, rows_last=rows_last)
    partials = pl.kernel(
        seg,
        out_type=jax.ShapeDtypeStruct((NC, n, h_dim), jnp.float32),
        mesh=mesh,
        scratch_types=[
            pltpu.VMEM((nch, CHUNK), jnp.int32),
            pltpu.VMEM((nch, CHUNK), jnp.int32),
            pltpu.VMEM((CHUNK, h_dim), jnp.float32),
            pltpu.VMEM((CHUNK, h_dim), jnp.float32),
            pltpu.VMEM((zchunk, h_dim), jnp.float32),
            pltpu.VMEM_SHARED((npad, h_dim), jnp.float32),
            pltpu.SemaphoreType.DMA,
            pltpu.SemaphoreType.DMA,
            pltpu.SemaphoreType.DMA,
            pltpu.SemaphoreType.DMA,
        ],
        compiler_params=pltpu.CompilerParams(use_tc_tiling_on_sc=False),
    )(h, src2, dstA)

    part_p = partials.reshape(NC, n // P, P * h_dim)
    out_p = pl.pallas_call(
        _tc2_body,
        out_shape=jax.ShapeDtypeStruct((n // P, P * h_dim), jnp.float32),
    )(h_p, part_p, Wrootbig, Webig, bconvbig)
    out = out_p.reshape(n, h_dim)

    mul = functools.partial(_sc_edgemul_body, nch=nch)
    emb = pl.kernel(
        mul,
        out_type=jax.ShapeDtypeStruct((chunks * CHUNK, h_dim), jnp.float32),
        mesh=mesh,
        scratch_types=[
            pltpu.VMEM((nch, CHUNK), jnp.int32),
            pltpu.VMEM((nch, CHUNK), jnp.int32),
            pltpu.VMEM((CHUNK, h_dim), jnp.float32),
            pltpu.VMEM((CHUNK, h_dim), jnp.float32),
            pltpu.VMEM((CHUNK, h_dim), jnp.float32),
            pltpu.VMEM((CHUNK, h_dim), jnp.float32),
            pltpu.SemaphoreType.DMA,
            pltpu.SemaphoreType.DMA,
            pltpu.SemaphoreType.DMA,
            pltpu.SemaphoreType.DMA,
            pltpu.SemaphoreType.DMA,
            pltpu.SemaphoreType.DMA,
        ],
        compiler_params=pltpu.CompilerParams(use_tc_tiling_on_sc=False),
    )(out, src2, dstB)

    ep = chunks * CHUNK // P
    emb_p = emb.reshape(ep, P * h_dim)
    blk = ep // 64
    score = pl.pallas_call(
        _tc3_body,
        grid=(ep // blk,),
        in_specs=[
            pl.BlockSpec((blk, P * h_dim), lambda i: (i, 0)),
            pl.BlockSpec((P * h_dim, P * 8), lambda i: (0, 0)),
            pl.BlockSpec((1, P * 8), lambda i: (0, 0)),
            pl.BlockSpec((P * 8, P), lambda i: (0, 0)),
            pl.BlockSpec((1, 1), lambda i: (0, 0)),
        ],
        out_specs=pl.BlockSpec((blk, P), lambda i: (i, 0)),
        out_shape=jax.ShapeDtypeStruct((ep, P), jnp.float32),
    )(emb_p, W1big, b1big, W2big, b2.reshape(1, 1))

    return score.reshape(-1)[:e]


# speedup vs baseline: 1.3595x; 1.3595x over previous
import functools

import jax
import jax.numpy as jnp
from jax import lax
from jax.experimental import pallas as pl
from jax.experimental.pallas import tpu as pltpu
from jax.experimental.pallas import tpu_sc as plsc

NC = 2
NS = 16
NW = NC * NS


def _tc1_body(x_ref, w0_ref, b0_ref, h_ref):
    h_ref[...] = jax.nn.relu(
        jnp.dot(x_ref[...], w0_ref[...], preferred_element_type=jnp.float32,
                precision=lax.Precision.HIGHEST)
        + b0_ref[...]
    )


def _tc2_body(h_ref, p_ref, wroot_ref, we_ref, bconv_ref, out_ref):
    a = p_ref[0] + p_ref[1]
    out_ref[...] = (
        jnp.dot(h_ref[...], wroot_ref[...], preferred_element_type=jnp.float32,
                precision=lax.Precision.HIGHEST)
        + jnp.dot(a, we_ref[...], preferred_element_type=jnp.float32,
                precision=lax.Precision.HIGHEST)
        + bconv_ref[...]
    )


def _tc3_body(emb_ref, w1_ref, b1_ref, w2_ref, b2_ref, out_ref):
    ee = jax.nn.relu(
        jnp.dot(emb_ref[...], w1_ref[...], preferred_element_type=jnp.float32)
        + b1_ref[...]
    )
    score = (
        jnp.dot(ee, w2_ref[...], preferred_element_type=jnp.float32)
        + b2_ref[...]
    )
    out_ref[...] = score


CHUNK = 128


def _sc_segsum_body(h_hbm, src2_hbm, dst2_hbm, part_hbm,
                    sidx_v, didx_v, rows0, rows1, zrow_v, acc_sh,
                    g0, g1, s0, s1,
                    *, nch, rows_per_sub, rows_last):
    cid = lax.axis_index("c")
    sid = lax.axis_index("s")
    wid = sid * NC + cid

    zchunk = zrow_v.shape[0]

    @pl.loop(0, zchunk)
    def _zero_buf(i):
        zrow_v[i, :] = jnp.zeros((16,), jnp.float32)

    @pl.loop(0, rows_per_sub // zchunk)
    def _zero_acc(k):
        pltpu.sync_copy(zrow_v, acc_sh.at[pl.ds(sid * rows_per_sub + k * zchunk, zchunk)])

    base = wid * nch
    pltpu.sync_copy(src2_hbm.at[pl.ds(base, nch)], sidx_v)
    pltpu.sync_copy(dst2_hbm.at[pl.ds(base, nch)], didx_v)
    plsc.subcore_barrier()

    rows = (rows0, rows1)
    gsem = (g0, g1)
    ssem = (s0, s1)
    pltpu.async_copy(h_hbm.at[sidx_v.at[0]], rows0, g0)

    @pl.loop(0, nch - 1, step=2)
    def _main(jj):
        for b in range(2):
            j = jj + b
            cur, oth = rows[b], rows[1 - b]
            pltpu.make_async_copy(h_hbm.at[sidx_v.at[j]], cur, gsem[b]).wait()
            pltpu.async_copy(cur, acc_sh.at[didx_v.at[j]], ssem[b], add=True)

            @pl.when(j >= 1)
            def _w():
                pltpu.make_async_copy(oth, acc_sh.at[didx_v.at[j]],
                                      ssem[1 - b]).wait()

            pltpu.async_copy(h_hbm.at[sidx_v.at[j + 1]], oth, gsem[1 - b])

    lb = (nch - 1) % 2
    cur, oth = rows[lb], rows[1 - lb]
    pltpu.make_async_copy(h_hbm.at[sidx_v.at[nch - 1]], cur, gsem[lb]).wait()
    pltpu.async_copy(cur, acc_sh.at[didx_v.at[nch - 1]], ssem[lb], add=True)
    pltpu.make_async_copy(oth, acc_sh.at[didx_v.at[nch - 1]], ssem[1 - lb]).wait()
    pltpu.make_async_copy(cur, acc_sh.at[didx_v.at[nch - 1]], ssem[lb]).wait()

    plsc.subcore_barrier()

    @pl.when(sid < NS - 1)
    def _full():
        pltpu.sync_copy(acc_sh.at[pl.ds(sid * rows_per_sub, rows_per_sub)],
                        part_hbm.at[cid, pl.ds(sid * rows_per_sub, rows_per_sub)])

    @pl.when(sid == NS - 1)
    def _clipped():
        pltpu.sync_copy(acc_sh.at[pl.ds((NS - 1) * rows_per_sub, rows_last)],
                        part_hbm.at[cid, pl.ds((NS - 1) * rows_per_sub, rows_last)])


def _sc_edgemul_body(out_hbm, src2_hbm, dst2_hbm, emb_hbm,
                     sidx_v, didx_v, sr0, sr1, dr0, dr1,
                     gs0, gs1, gd0, gd1, w0, w1,
                     *, nch):
    cid = lax.axis_index("c")
    sid = lax.axis_index("s")
    wid = sid * NC + cid

    base = wid * nch
    pltpu.sync_copy(src2_hbm.at[pl.ds(base, nch)], sidx_v)
    pltpu.sync_copy(dst2_hbm.at[pl.ds(base, nch)], didx_v)

    srows = (sr0, sr1)
    drows = (dr0, dr1)
    gsS = (gs0, gs1)
    gsD = (gd0, gd1)
    wsem = (w0, w1)

    def _wr_dst(j):
        return emb_hbm.at[pl.ds((base + j) * CHUNK, CHUNK)]

    pltpu.async_copy(out_hbm.at[sidx_v.at[0]], sr0, gs0)
    pltpu.async_copy(out_hbm.at[didx_v.at[0]], dr0, gd0)

    def _process(j, b):
        cur_s, cur_d = srows[b], drows[b]
        pltpu.make_async_copy(out_hbm.at[sidx_v.at[j]], cur_s, gsS[b]).wait()
        pltpu.make_async_copy(out_hbm.at[didx_v.at[j]], cur_d, gsD[b]).wait()

        @pl.loop(0, CHUNK, unroll=8)
        def _mul(r):
            cur_s[r, :] = cur_s[r, :] * cur_d[r, :]

        pltpu.async_copy(cur_s, _wr_dst(j), wsem[b])

    @pl.loop(0, nch - 1, step=2)
    def _main(jj):
        for b in range(2):
            j = jj + b
            _process(j, b)

            @pl.when(j >= 1)
            def _w():
                pltpu.make_async_copy(srows[1 - b], _wr_dst(j), wsem[1 - b]).wait()

            pltpu.async_copy(out_hbm.at[sidx_v.at[j + 1]], srows[1 - b], gsS[1 - b])
            pltpu.async_copy(out_hbm.at[didx_v.at[j + 1]], drows[1 - b], gsD[1 - b])

    lb = (nch - 1) % 2
    _process(nch - 1, lb)
    pltpu.make_async_copy(srows[1 - lb], _wr_dst(nch - 1), wsem[1 - lb]).wait()
    pltpu.make_async_copy(srows[lb], _wr_dst(nch - 1), wsem[lb]).wait()


def kernel(x, edge_index, W0, b0, We1, be1, We2, be2, Wroot, bconv, W1, b1,
           W2, b2):
    n, d = x.shape
    e = edge_index.shape[1]
    h_dim = W0.shape[1]

    src = edge_index[0]
    dst = edge_index[1]

    e1 = jax.nn.relu(We1[0] + be1)
    w_e = (e1 @ We2 + be2).reshape(h_dim, h_dim)

    P = 128 // h_dim
    eyeP = jnp.eye(P, dtype=jnp.float32)
    W0big = jnp.kron(eyeP, W0)
    b0big = jnp.tile(b0, P).reshape(1, P * h_dim)
    Wrootbig = jnp.kron(eyeP, Wroot)
    Webig = jnp.kron(eyeP, w_e)
    bconvbig = jnp.tile(bconv, P).reshape(1, P * h_dim)
    W1big = jnp.kron(eyeP, W1)
    b1big = jnp.tile(b1, P).reshape(1, P * 8)
    W2big = jnp.kron(eyeP, W2)

    h_p = pl.pallas_call(
        _tc1_body,
        out_shape=jax.ShapeDtypeStruct((n // P, P * h_dim), jnp.float32),
    )(x.reshape(n // P, P * d), W0big, b0big)
    h = h_p.reshape(n, h_dim)

    chunks = -(-e // (CHUNK * NW)) * NW
    nch = chunks // NW
    pad_e = chunks * CHUNK - e
    src2 = jnp.concatenate(
        [src, jnp.zeros((pad_e,), jnp.int32)]).reshape(chunks, CHUNK)
    dstA = jnp.concatenate(
        [dst, jnp.full((pad_e,), n, jnp.int32)]).reshape(chunks, CHUNK)
    dstB = jnp.concatenate(
        [dst, jnp.zeros((pad_e,), jnp.int32)]).reshape(chunks, CHUNK)
    rows_per_sub = -(-n // (8 * NS)) * 8
    npad = rows_per_sub * NS
    zchunk = rows_per_sub // 4
    mesh = plsc.VectorSubcoreMesh(core_axis_name="c", subcore_axis_name="s",
                                  num_cores=NC, num_subcores=NS)
    rows_last = n - rows_per_sub * (NS - 1)
    seg = functools.partial(_sc_segsum_body, nch=nch,
                            rows_per_sub=rows_per_sub, rows_last=rows_last)
    partials = pl.kernel(
        seg,
        out_type=jax.ShapeDtypeStruct((NC, n, h_dim), jnp.float32),
        mesh=mesh,
        scratch_types=[
            pltpu.VMEM((nch, CHUNK), jnp.int32),
            pltpu.VMEM((nch, CHUNK), jnp.int32),
            pltpu.VMEM((CHUNK, h_dim), jnp.float32),
            pltpu.VMEM((CHUNK, h_dim), jnp.float32),
            pltpu.VMEM((zchunk, h_dim), jnp.float32),
            pltpu.VMEM_SHARED((npad, h_dim), jnp.float32),
            pltpu.SemaphoreType.DMA,
            pltpu.SemaphoreType.DMA,
            pltpu.SemaphoreType.DMA,
            pltpu.SemaphoreType.DMA,
        ],
        compiler_params=pltpu.CompilerParams(use_tc_tiling_on_sc=False),
    )(h, src2, dstA)

    part_p = partials.reshape(NC, n // P, P * h_dim)
    out_p = pl.pallas_call(
        _tc2_body,
        out_shape=jax.ShapeDtypeStruct((n // P, P * h_dim), jnp.float32),
    )(h_p, part_p, Wrootbig, Webig, bconvbig)
    out = out_p.reshape(n, h_dim)

    mul = functools.partial(_sc_edgemul_body, nch=nch)
    emb = pl.kernel(
        mul,
        out_type=jax.ShapeDtypeStruct((chunks * CHUNK, h_dim), jnp.float32),
        mesh=mesh,
        scratch_types=[
            pltpu.VMEM((nch, CHUNK), jnp.int32),
            pltpu.VMEM((nch, CHUNK), jnp.int32),
            pltpu.VMEM((CHUNK, h_dim), jnp.float32),
            pltpu.VMEM((CHUNK, h_dim), jnp.float32),
            pltpu.VMEM((CHUNK, h_dim), jnp.float32),
            pltpu.VMEM((CHUNK, h_dim), jnp.float32),
            pltpu.SemaphoreType.DMA,
            pltpu.SemaphoreType.DMA,
            pltpu.SemaphoreType.DMA,
            pltpu.SemaphoreType.DMA,
            pltpu.SemaphoreType.DMA,
            pltpu.SemaphoreType.DMA,
        ],
        compiler_params=pltpu.CompilerParams(use_tc_tiling_on_sc=False),
    )(out, src2, dstB)

    ep = chunks * CHUNK // P
    emb_p = emb.reshape(ep, P * h_dim)
    blk = ep // 4
    score = pl.pallas_call(
        _tc3_body,
        grid=(ep // blk,),
        in_specs=[
            pl.BlockSpec((blk, P * h_dim), lambda i: (i, 0)),
            pl.BlockSpec((P * h_dim, P * 8), lambda i: (0, 0)),
            pl.BlockSpec((1, P * 8), lambda i: (0, 0)),
            pl.BlockSpec((P * 8, P), lambda i: (0, 0)),
            pl.BlockSpec((1, 1), lambda i: (0, 0)),
        ],
        out_specs=pl.BlockSpec((blk, P), lambda i: (i, 0)),
        out_shape=jax.ShapeDtypeStruct((ep, P), jnp.float32),
    )(emb_p, W1big, b1big, W2big, b2.reshape(1, 1))

    return score.reshape(-1)[:e]


# speedup vs baseline: 1.8347x; 1.3495x over previous
import functools

import jax
import jax.numpy as jnp
from jax import lax
from jax.experimental import pallas as pl
from jax.experimental.pallas import tpu as pltpu
from jax.experimental.pallas import tpu_sc as plsc

NC = 2
NS = 16
NW = NC * NS


def _tc1_body(x_ref, w0_ref, b0_ref, h_ref):
    h_ref[...] = jax.nn.relu(
        jnp.dot(x_ref[...], w0_ref[...], preferred_element_type=jnp.float32,
                precision=lax.Precision.HIGHEST)
        + b0_ref[...]
    )


def _tc2_body(h_ref, p_ref, wroot_ref, we_ref, bconv_ref, out_ref):
    a = p_ref[0] + p_ref[1]
    out_ref[...] = (
        jnp.dot(h_ref[...], wroot_ref[...], preferred_element_type=jnp.float32,
                precision=lax.Precision.HIGHEST)
        + jnp.dot(a, we_ref[...], preferred_element_type=jnp.float32,
                precision=lax.Precision.HIGHEST)
        + bconv_ref[...]
    )


def _tc3_body(emb_ref, w1_ref, b1_ref, w2_ref, b2_ref, out_ref):
    ee = jax.nn.relu(
        jnp.dot(emb_ref[...], w1_ref[...], preferred_element_type=jnp.float32)
        + b1_ref[...]
    )
    score = (
        jnp.dot(ee, w2_ref[...], preferred_element_type=jnp.float32)
        + b2_ref[...]
    )
    out_ref[...] = score


CHUNK = 128


def _sc_segsum_body(h_hbm, src2_hbm, dst2_hbm, part_hbm,
                    sidx_v, didx_v, rows0, rows1, rows2, rows3, zrow_v, acc_sh,
                    g0, g1, g2, g3, s0, s1, s2, s3,
                    *, nch, rows_per_sub, rows_last):
    cid = lax.axis_index("c")
    sid = lax.axis_index("s")
    wid = sid * NC + cid

    zchunk = zrow_v.shape[0]

    @pl.loop(0, zchunk)
    def _zero_buf(i):
        zrow_v[i, :] = jnp.zeros((16,), jnp.float32)

    @pl.loop(0, rows_per_sub // zchunk)
    def _zero_acc(k):
        pltpu.sync_copy(zrow_v, acc_sh.at[pl.ds(sid * rows_per_sub + k * zchunk, zchunk)])

    base = wid * nch
    pltpu.sync_copy(src2_hbm.at[pl.ds(base, nch)], sidx_v)
    pltpu.sync_copy(dst2_hbm.at[pl.ds(base, nch)], didx_v)
    plsc.subcore_barrier()

    rows = (rows0, rows1, rows2, rows3)
    gsem = (g0, g1, g2, g3)
    ssem = (s0, s1, s2, s3)

    def _wait_gather(j, b):
        pltpu.make_async_copy(h_hbm.at[sidx_v.at[j]], rows[b], gsem[b]).wait()

    def _wait_scatter(j, b):
        pltpu.make_async_copy(rows[b], acc_sh.at[didx_v.at[j]], ssem[b]).wait()

    def _step(j, b):
        _wait_gather(j, b)
        pltpu.async_copy(rows[b], acc_sh.at[didx_v.at[j]], ssem[b], add=True)
        b2 = (b + 2) % 4

        @pl.when(j >= 2)
        def _w():
            _wait_scatter(j, b2)

        pltpu.async_copy(h_hbm.at[sidx_v.at[j + 2]], rows[b2], gsem[b2])

    pltpu.async_copy(h_hbm.at[sidx_v.at[0]], rows0, g0)
    pltpu.async_copy(h_hbm.at[sidx_v.at[1]], rows1, g1)

    main_end = ((nch - 3) // 4) * 4

    @pl.loop(0, main_end, step=4)
    def _main(jj):
        for b in range(4):
            _step(jj + b, b)

    for j in range(main_end, nch):
        b = j % 4
        _wait_gather(j, b)
        pltpu.async_copy(rows[b], acc_sh.at[didx_v.at[j]], ssem[b], add=True)
        _wait_scatter(j, (j + 2) % 4)
        if j + 2 < nch:
            b2 = (j + 2) % 4
            pltpu.async_copy(h_hbm.at[sidx_v.at[j + 2]], rows[b2], gsem[b2])
    _wait_scatter(nch - 2, (nch - 2) % 4)
    _wait_scatter(nch - 1, (nch - 1) % 4)

    plsc.subcore_barrier()

    @pl.when(sid < NS - 1)
    def _full():
        pltpu.sync_copy(acc_sh.at[pl.ds(sid * rows_per_sub, rows_per_sub)],
                        part_hbm.at[cid, pl.ds(sid * rows_per_sub, rows_per_sub)])

    @pl.when(sid == NS - 1)
    def _clipped():
        pltpu.sync_copy(acc_sh.at[pl.ds((NS - 1) * rows_per_sub, rows_last)],
                        part_hbm.at[cid, pl.ds((NS - 1) * rows_per_sub, rows_last)])


def _sc_edgemul_body(out_hbm, src2_hbm, dst2_hbm, emb_hbm,
                     sidx_v, didx_v, sr0, sr1, sr2, sr3, dr0, dr1, dr2, dr3,
                     gs0, gs1, gs2, gs3, gd0, gd1, gd2, gd3, w0, w1, w2, w3,
                     *, nch):
    cid = lax.axis_index("c")
    sid = lax.axis_index("s")
    wid = sid * NC + cid

    base = wid * nch
    pltpu.sync_copy(src2_hbm.at[pl.ds(base, nch)], sidx_v)
    pltpu.sync_copy(dst2_hbm.at[pl.ds(base, nch)], didx_v)

    srows = (sr0, sr1, sr2, sr3)
    drows = (dr0, dr1, dr2, dr3)
    gsS = (gs0, gs1, gs2, gs3)
    gsD = (gd0, gd1, gd2, gd3)
    wsem = (w0, w1, w2, w3)

    def _wr_dst(j):
        return emb_hbm.at[pl.ds((base + j) * CHUNK, CHUNK)]

    def _start_gathers(j, b):
        pltpu.async_copy(out_hbm.at[sidx_v.at[j]], srows[b], gsS[b])
        pltpu.async_copy(out_hbm.at[didx_v.at[j]], drows[b], gsD[b])

    def _wait_writeout(j, b):
        pltpu.make_async_copy(srows[b], _wr_dst(j), wsem[b]).wait()

    def _process(j, b):
        cur_s, cur_d = srows[b], drows[b]
        pltpu.make_async_copy(out_hbm.at[sidx_v.at[j]], cur_s, gsS[b]).wait()
        pltpu.make_async_copy(out_hbm.at[didx_v.at[j]], cur_d, gsD[b]).wait()

        @pl.loop(0, CHUNK, unroll=8)
        def _mul(r):
            cur_s[r, :] = cur_s[r, :] * cur_d[r, :]

        pltpu.async_copy(cur_s, _wr_dst(j), wsem[b])

    _start_gathers(0, 0)
    _start_gathers(1, 1)

    main_end = ((nch - 3) // 4) * 4

    @pl.loop(0, main_end, step=4)
    def _main(jj):
        for b in range(4):
            j = jj + b
            _process(j, b)
            b2 = (b + 2) % 4

            @pl.when(j >= 2)
            def _w():
                _wait_writeout(j, b2)

            _start_gathers(j + 2, b2)

    for j in range(main_end, nch):
        b = j % 4
        _process(j, b)
        _wait_writeout(j, (j + 2) % 4)
        if j + 2 < nch:
            _start_gathers(j + 2, (j + 2) % 4)
    _wait_writeout(nch - 2, (nch - 2) % 4)
    _wait_writeout(nch - 1, (nch - 1) % 4)


def kernel(x, edge_index, W0, b0, We1, be1, We2, be2, Wroot, bconv, W1, b1,
           W2, b2):
    n, d = x.shape
    e = edge_index.shape[1]
    h_dim = W0.shape[1]

    src = edge_index[0]
    dst = edge_index[1]

    e1 = jax.nn.relu(We1[0] + be1)
    w_e = (e1 @ We2 + be2).reshape(h_dim, h_dim)

    P = 128 // h_dim
    eyeP = jnp.eye(P, dtype=jnp.float32)
    W0big = jnp.kron(eyeP, W0)
    b0big = jnp.tile(b0, P).reshape(1, P * h_dim)
    Wrootbig = jnp.kron(eyeP, Wroot)
    Webig = jnp.kron(eyeP, w_e)
    bconvbig = jnp.tile(bconv, P).reshape(1, P * h_dim)
    W1big = jnp.kron(eyeP, W1)
    b1big = jnp.tile(b1, P).reshape(1, P * 8)
    W2big = jnp.kron(eyeP, W2)

    h_p = pl.pallas_call(
        _tc1_body,
        out_shape=jax.ShapeDtypeStruct((n // P, P * h_dim), jnp.float32),
    )(x.reshape(n // P, P * d), W0big, b0big)
    h = h_p.reshape(n, h_dim)

    chunks = -(-e // (CHUNK * NW)) * NW
    nch = chunks // NW
    pad_e = chunks * CHUNK - e
    src2 = jnp.concatenate(
        [src, jnp.zeros((pad_e,), jnp.int32)]).reshape(chunks, CHUNK)
    dstA = jnp.concatenate(
        [dst, jnp.full((pad_e,), n, jnp.int32)]).reshape(chunks, CHUNK)
    dstB = jnp.concatenate(
        [dst, jnp.zeros((pad_e,), jnp.int32)]).reshape(chunks, CHUNK)
    rows_per_sub = -(-n // (8 * NS)) * 8
    npad = rows_per_sub * NS
    zchunk = rows_per_sub // 4
    mesh = plsc.VectorSubcoreMesh(core_axis_name="c", subcore_axis_name="s",
                                  num_cores=NC, num_subcores=NS)
    rows_last = n - rows_per_sub * (NS - 1)
    seg = functools.partial(_sc_segsum_body, nch=nch,
                            rows_per_sub=rows_per_sub, rows_last=rows_last)
    partials = pl.kernel(
        seg,
        out_type=jax.ShapeDtypeStruct((NC, n, h_dim), jnp.float32),
        mesh=mesh,
        scratch_types=(
            [pltpu.VMEM((nch, CHUNK), jnp.int32)] * 2
            + [pltpu.VMEM((CHUNK, h_dim), jnp.float32)] * 4
            + [pltpu.VMEM((zchunk, h_dim), jnp.float32),
               pltpu.VMEM_SHARED((npad, h_dim), jnp.float32)]
            + [pltpu.SemaphoreType.DMA] * 8
        ),
        compiler_params=pltpu.CompilerParams(use_tc_tiling_on_sc=False),
    )(h, src2, dstA)

    part_p = partials.reshape(NC, n // P, P * h_dim)
    out_p = pl.pallas_call(
        _tc2_body,
        out_shape=jax.ShapeDtypeStruct((n // P, P * h_dim), jnp.float32),
    )(h_p, part_p, Wrootbig, Webig, bconvbig)
    out = out_p.reshape(n, h_dim)

    mul = functools.partial(_sc_edgemul_body, nch=nch)
    emb = pl.kernel(
        mul,
        out_type=jax.ShapeDtypeStruct((chunks * CHUNK, h_dim), jnp.float32),
        mesh=mesh,
        scratch_types=(
            [pltpu.VMEM((nch, CHUNK), jnp.int32)] * 2
            + [pltpu.VMEM((CHUNK, h_dim), jnp.float32)] * 8
            + [pltpu.SemaphoreType.DMA] * 12
        ),
        compiler_params=pltpu.CompilerParams(use_tc_tiling_on_sc=False),
    )(out, src2, dstB)

    ep = chunks * CHUNK // P
    emb_p = emb.reshape(ep, P * h_dim)
    blk = ep // 4
    score = pl.pallas_call(
        _tc3_body,
        grid=(ep // blk,),
        in_specs=[
            pl.BlockSpec((blk, P * h_dim), lambda i: (i, 0)),
            pl.BlockSpec((P * h_dim, P * 8), lambda i: (0, 0)),
            pl.BlockSpec((1, P * 8), lambda i: (0, 0)),
            pl.BlockSpec((P * 8, P), lambda i: (0, 0)),
            pl.BlockSpec((1, 1), lambda i: (0, 0)),
        ],
        out_specs=pl.BlockSpec((blk, P), lambda i: (i, 0)),
        out_shape=jax.ShapeDtypeStruct((ep, P), jnp.float32),
    )(emb_p, W1big, b1big, W2big, b2.reshape(1, 1))

    return score.reshape(-1)[:e]


# speedup vs baseline: 1.9125x; 1.0424x over previous
import functools

import jax
import jax.numpy as jnp
from jax import lax
from jax.experimental import pallas as pl
from jax.experimental.pallas import tpu as pltpu
from jax.experimental.pallas import tpu_sc as plsc

NC = 2
NS = 16
NW = NC * NS


def _tc1_body(x_ref, w0_ref, b0_ref, h_ref):
    h_ref[...] = jax.nn.relu(
        jnp.dot(x_ref[...], w0_ref[...], preferred_element_type=jnp.float32,
                precision=lax.Precision.HIGHEST)
        + b0_ref[...]
    )


def _tc2_body(h_ref, p_ref, wroot_ref, we_ref, bconv_ref, out_ref):
    a = p_ref[0] + p_ref[1]
    out_ref[...] = (
        jnp.dot(h_ref[...], wroot_ref[...], preferred_element_type=jnp.float32,
                precision=lax.Precision.HIGHEST)
        + jnp.dot(a, we_ref[...], preferred_element_type=jnp.float32,
                precision=lax.Precision.HIGHEST)
        + bconv_ref[...]
    )


def _tc3_body(emb_ref, w1_ref, b1_ref, w2_ref, b2_ref, out_ref):
    ee = jax.nn.relu(
        jnp.dot(emb_ref[...], w1_ref[...], preferred_element_type=jnp.float32)
        + b1_ref[...]
    )
    score = (
        jnp.dot(ee, w2_ref[...], preferred_element_type=jnp.float32)
        + b2_ref[...]
    )
    out_ref[...] = score


CHUNK = 128


NBUF = 8
PF = 4


def _sc_segsum_body(h_hbm, src2_hbm, dst2_hbm, part_hbm, *sc,
                    nch, rows_per_sub, rows_last):
    sidx_v, didx_v = sc[0], sc[1]
    rows = sc[2:2 + NBUF]
    zrow_v, acc_sh = sc[2 + NBUF], sc[3 + NBUF]
    gsem = sc[4 + NBUF:4 + 2 * NBUF]
    ssem = sc[4 + 2 * NBUF:4 + 3 * NBUF]
    cid = lax.axis_index("c")
    sid = lax.axis_index("s")
    wid = sid * NC + cid

    zchunk = zrow_v.shape[0]

    @pl.loop(0, zchunk)
    def _zero_buf(i):
        zrow_v[i, :] = jnp.zeros((16,), jnp.float32)

    @pl.loop(0, rows_per_sub // zchunk)
    def _zero_acc(k):
        pltpu.sync_copy(zrow_v, acc_sh.at[pl.ds(sid * rows_per_sub + k * zchunk, zchunk)])

    base = wid * nch
    pltpu.sync_copy(src2_hbm.at[pl.ds(base, nch)], sidx_v)
    pltpu.sync_copy(dst2_hbm.at[pl.ds(base, nch)], didx_v)
    plsc.subcore_barrier()

    def _wait_gather(j, b):
        pltpu.make_async_copy(h_hbm.at[sidx_v.at[j]], rows[b], gsem[b]).wait()

    def _wait_scatter(j, b):
        pltpu.make_async_copy(rows[b], acc_sh.at[didx_v.at[j]], ssem[b]).wait()

    def _step(j, b):
        _wait_gather(j, b)
        pltpu.async_copy(rows[b], acc_sh.at[didx_v.at[j]], ssem[b], add=True)
        b2 = (b + PF) % NBUF

        @pl.when(j >= PF)
        def _w():
            _wait_scatter(j, b2)

        pltpu.async_copy(h_hbm.at[sidx_v.at[j + PF]], rows[b2], gsem[b2])

    for k in range(PF):
        pltpu.async_copy(h_hbm.at[sidx_v.at[k]], rows[k], gsem[k])

    main_end = ((nch - PF - 1) // NBUF) * NBUF

    @pl.loop(0, main_end, step=NBUF)
    def _main(jj):
        for b in range(NBUF):
            _step(jj + b, b)

    for j in range(main_end, nch):
        b = j % NBUF
        _wait_gather(j, b)
        pltpu.async_copy(rows[b], acc_sh.at[didx_v.at[j]], ssem[b], add=True)
        if j >= PF:
            _wait_scatter(j, (j + PF) % NBUF)
        if j + PF < nch:
            b2 = (j + PF) % NBUF
            pltpu.async_copy(h_hbm.at[sidx_v.at[j + PF]], rows[b2], gsem[b2])
    for j in range(max(0, nch - PF), nch):
        _wait_scatter(j, j % NBUF)

    plsc.subcore_barrier()

    @pl.when(sid < NS - 1)
    def _full():
        pltpu.sync_copy(acc_sh.at[pl.ds(sid * rows_per_sub, rows_per_sub)],
                        part_hbm.at[cid, pl.ds(sid * rows_per_sub, rows_per_sub)])

    @pl.when(sid == NS - 1)
    def _clipped():
        pltpu.sync_copy(acc_sh.at[pl.ds((NS - 1) * rows_per_sub, rows_last)],
                        part_hbm.at[cid, pl.ds((NS - 1) * rows_per_sub, rows_last)])


def _sc_edgemul_body(out_hbm, src2_hbm, dst2_hbm, emb_hbm, *sc, nch):
    sidx_v, didx_v = sc[0], sc[1]
    srows = sc[2:2 + NBUF]
    drows = sc[2 + NBUF:2 + 2 * NBUF]
    gsS = sc[2 + 2 * NBUF:2 + 3 * NBUF]
    gsD = sc[2 + 3 * NBUF:2 + 4 * NBUF]
    wsem = sc[2 + 4 * NBUF:2 + 5 * NBUF]
    cid = lax.axis_index("c")
    sid = lax.axis_index("s")
    wid = sid * NC + cid

    base = wid * nch
    pltpu.sync_copy(src2_hbm.at[pl.ds(base, nch)], sidx_v)
    pltpu.sync_copy(dst2_hbm.at[pl.ds(base, nch)], didx_v)

    def _wr_dst(j):
        return emb_hbm.at[pl.ds((base + j) * CHUNK, CHUNK)]

    def _start_gathers(j, b):
        pltpu.async_copy(out_hbm.at[sidx_v.at[j]], srows[b], gsS[b])
        pltpu.async_copy(out_hbm.at[didx_v.at[j]], drows[b], gsD[b])

    def _wait_writeout(j, b):
        pltpu.make_async_copy(srows[b], _wr_dst(j), wsem[b]).wait()

    def _process(j, b):
        cur_s, cur_d = srows[b], drows[b]
        pltpu.make_async_copy(out_hbm.at[sidx_v.at[j]], cur_s, gsS[b]).wait()
        pltpu.make_async_copy(out_hbm.at[didx_v.at[j]], cur_d, gsD[b]).wait()

        @pl.loop(0, CHUNK, unroll=8)
        def _mul(r):
            cur_s[r, :] = cur_s[r, :] * cur_d[r, :]

        pltpu.async_copy(cur_s, _wr_dst(j), wsem[b])

    for k in range(PF):
        _start_gathers(k, k)

    main_end = ((nch - PF - 1) // NBUF) * NBUF

    @pl.loop(0, main_end, step=NBUF)
    def _main(jj):
        for b in range(NBUF):
            j = jj + b
            _process(j, b)
            b2 = (b + PF) % NBUF

            @pl.when(j >= PF)
            def _w():
                _wait_writeout(j, b2)

            _start_gathers(j + PF, b2)

    for j in range(main_end, nch):
        b = j % NBUF
        _process(j, b)
        if j >= PF:
            _wait_writeout(j, (j + PF) % NBUF)
        if j + PF < nch:
            _start_gathers(j + PF, (j + PF) % NBUF)
    for j in range(max(0, nch - PF), nch):
        _wait_writeout(j, j % NBUF)


def kernel(x, edge_index, W0, b0, We1, be1, We2, be2, Wroot, bconv, W1, b1,
           W2, b2):
    n, d = x.shape
    e = edge_index.shape[1]
    h_dim = W0.shape[1]

    src = edge_index[0]
    dst = edge_index[1]

    e1 = jax.nn.relu(We1[0] + be1)
    w_e = (e1 @ We2 + be2).reshape(h_dim, h_dim)

    P = 128 // h_dim
    eyeP = jnp.eye(P, dtype=jnp.float32)
    W0big = jnp.kron(eyeP, W0)
    b0big = jnp.tile(b0, P).reshape(1, P * h_dim)
    Wrootbig = jnp.kron(eyeP, Wroot)
    Webig = jnp.kron(eyeP, w_e)
    bconvbig = jnp.tile(bconv, P).reshape(1, P * h_dim)
    W1big = jnp.kron(eyeP, W1)
    b1big = jnp.tile(b1, P).reshape(1, P * 8)
    W2big = jnp.kron(eyeP, W2)

    h_p = pl.pallas_call(
        _tc1_body,
        out_shape=jax.ShapeDtypeStruct((n // P, P * h_dim), jnp.float32),
    )(x.reshape(n // P, P * d), W0big, b0big)
    h = h_p.reshape(n, h_dim)

    chunks = -(-e // (CHUNK * NW)) * NW
    nch = chunks // NW
    pad_e = chunks * CHUNK - e
    src2 = jnp.concatenate(
        [src, jnp.zeros((pad_e,), jnp.int32)]).reshape(chunks, CHUNK)
    dstA = jnp.concatenate(
        [dst, jnp.full((pad_e,), n, jnp.int32)]).reshape(chunks, CHUNK)
    dstB = jnp.concatenate(
        [dst, jnp.zeros((pad_e,), jnp.int32)]).reshape(chunks, CHUNK)
    rows_per_sub = -(-n // (8 * NS)) * 8
    npad = rows_per_sub * NS
    zchunk = rows_per_sub // 4
    mesh = plsc.VectorSubcoreMesh(core_axis_name="c", subcore_axis_name="s",
                                  num_cores=NC, num_subcores=NS)
    rows_last = n - rows_per_sub * (NS - 1)
    seg = functools.partial(_sc_segsum_body, nch=nch,
                            rows_per_sub=rows_per_sub, rows_last=rows_last)
    partials = pl.kernel(
        seg,
        out_type=jax.ShapeDtypeStruct((NC, n, h_dim), jnp.float32),
        mesh=mesh,
        scratch_types=(
            [pltpu.VMEM((nch, CHUNK), jnp.int32)] * 2
            + [pltpu.VMEM((CHUNK, h_dim), jnp.float32)] * NBUF
            + [pltpu.VMEM((zchunk, h_dim), jnp.float32),
               pltpu.VMEM_SHARED((npad, h_dim), jnp.float32)]
            + [pltpu.SemaphoreType.DMA] * (2 * NBUF)
        ),
        compiler_params=pltpu.CompilerParams(use_tc_tiling_on_sc=False),
    )(h, src2, dstA)

    part_p = partials.reshape(NC, n // P, P * h_dim)
    out_p = pl.pallas_call(
        _tc2_body,
        out_shape=jax.ShapeDtypeStruct((n // P, P * h_dim), jnp.float32),
    )(h_p, part_p, Wrootbig, Webig, bconvbig)
    out = out_p.reshape(n, h_dim)

    mul = functools.partial(_sc_edgemul_body, nch=nch)
    emb = pl.kernel(
        mul,
        out_type=jax.ShapeDtypeStruct((chunks * CHUNK, h_dim), jnp.float32),
        mesh=mesh,
        scratch_types=(
            [pltpu.VMEM((nch, CHUNK), jnp.int32)] * 2
            + [pltpu.VMEM((CHUNK, h_dim), jnp.float32)] * (2 * NBUF)
            + [pltpu.SemaphoreType.DMA] * (3 * NBUF)
        ),
        compiler_params=pltpu.CompilerParams(use_tc_tiling_on_sc=False),
    )(out, src2, dstB)

    ep = chunks * CHUNK // P
    emb_p = emb.reshape(ep, P * h_dim)
    blk = ep // 4
    score = pl.pallas_call(
        _tc3_body,
        grid=(ep // blk,),
        in_specs=[
            pl.BlockSpec((blk, P * h_dim), lambda i: (i, 0)),
            pl.BlockSpec((P * h_dim, P * 8), lambda i: (0, 0)),
            pl.BlockSpec((1, P * 8), lambda i: (0, 0)),
            pl.BlockSpec((P * 8, P), lambda i: (0, 0)),
            pl.BlockSpec((1, 1), lambda i: (0, 0)),
        ],
        out_specs=pl.BlockSpec((blk, P), lambda i: (i, 0)),
        out_shape=jax.ShapeDtypeStruct((ep, P), jnp.float32),
    )(emb_p, W1big, b1big, W2big, b2.reshape(1, 1))

    return score.reshape(-1)[:e]


# speedup vs baseline: 2.0616x; 1.0779x over previous
import functools

import jax
import jax.numpy as jnp
from jax import lax
from jax.experimental import pallas as pl
from jax.experimental.pallas import tpu as pltpu
from jax.experimental.pallas import tpu_sc as plsc

NC = 2
NS = 16
NW = NC * NS


def _tc1_body(x_ref, w0_ref, b0_ref, h_ref):
    h_ref[...] = jax.nn.relu(
        jnp.dot(x_ref[...], w0_ref[...], preferred_element_type=jnp.float32,
                precision=lax.Precision.HIGHEST)
        + b0_ref[...]
    )


def _tc2_body(h_ref, p_ref, wroot_ref, we_ref, bconv_ref, out_ref):
    a = p_ref[0] + p_ref[1]
    out_ref[...] = (
        jnp.dot(h_ref[...], wroot_ref[...], preferred_element_type=jnp.float32,
                precision=lax.Precision.HIGHEST)
        + jnp.dot(a, we_ref[...], preferred_element_type=jnp.float32,
                precision=lax.Precision.HIGHEST)
        + bconv_ref[...]
    )


def _tc3_body(emb_ref, w1_ref, b1_ref, w2_ref, b2_ref, out_ref):
    ee = jax.nn.relu(
        jnp.dot(emb_ref[...], w1_ref[...], preferred_element_type=jnp.float32)
        + b1_ref[...]
    )
    score_t = lax.dot_general(w2_ref[...], ee, (((0,), (1,)), ((), ())),
                              preferred_element_type=jnp.float32)
    out_ref[...] = score_t + b2_ref[...]


CHUNK = 128


NBUF = 8
PF = 4


def _sc_segsum_body(h_hbm, src2_hbm, dst2_hbm, part_hbm, *sc,
                    nch, rows_per_sub, rows_last):
    sidx_v, didx_v = sc[0], sc[1]
    rows = sc[2:2 + NBUF]
    zrow_v, acc_sh = sc[2 + NBUF], sc[3 + NBUF]
    gsem = sc[4 + NBUF:4 + 2 * NBUF]
    ssem = sc[4 + 2 * NBUF:4 + 3 * NBUF]
    cid = lax.axis_index("c")
    sid = lax.axis_index("s")
    wid = sid * NC + cid

    zchunk = zrow_v.shape[0]

    @pl.loop(0, zchunk)
    def _zero_buf(i):
        zrow_v[i, :] = jnp.zeros((16,), jnp.float32)

    @pl.loop(0, rows_per_sub // zchunk)
    def _zero_acc(k):
        pltpu.sync_copy(zrow_v, acc_sh.at[pl.ds(sid * rows_per_sub + k * zchunk, zchunk)])

    base = wid * nch
    pltpu.sync_copy(src2_hbm.at[pl.ds(base, nch)], sidx_v)
    pltpu.sync_copy(dst2_hbm.at[pl.ds(base, nch)], didx_v)
    plsc.subcore_barrier()

    def _wait_gather(j, b):
        pltpu.make_async_copy(h_hbm.at[sidx_v.at[j]], rows[b], gsem[b]).wait()

    def _wait_scatter(j, b):
        pltpu.make_async_copy(rows[b], acc_sh.at[didx_v.at[j]], ssem[b]).wait()

    def _step(j, b):
        _wait_gather(j, b)
        pltpu.async_copy(rows[b], acc_sh.at[didx_v.at[j]], ssem[b], add=True)
        b2 = (b + PF) % NBUF

        @pl.when(j >= PF)
        def _w():
            _wait_scatter(j, b2)

        pltpu.async_copy(h_hbm.at[sidx_v.at[j + PF]], rows[b2], gsem[b2])

    for k in range(PF):
        pltpu.async_copy(h_hbm.at[sidx_v.at[k]], rows[k], gsem[k])

    main_end = ((nch - PF - 1) // NBUF) * NBUF

    @pl.loop(0, main_end, step=NBUF)
    def _main(jj):
        for b in range(NBUF):
            _step(jj + b, b)

    for j in range(main_end, nch):
        b = j % NBUF
        _wait_gather(j, b)
        pltpu.async_copy(rows[b], acc_sh.at[didx_v.at[j]], ssem[b], add=True)
        if j >= PF:
            _wait_scatter(j, (j + PF) % NBUF)
        if j + PF < nch:
            b2 = (j + PF) % NBUF
            pltpu.async_copy(h_hbm.at[sidx_v.at[j + PF]], rows[b2], gsem[b2])
    for j in range(max(0, nch - PF), nch):
        _wait_scatter(j, j % NBUF)

    plsc.subcore_barrier()

    @pl.when(sid < NS - 1)
    def _full():
        pltpu.sync_copy(acc_sh.at[pl.ds(sid * rows_per_sub, rows_per_sub)],
                        part_hbm.at[cid, pl.ds(sid * rows_per_sub, rows_per_sub)])

    @pl.when(sid == NS - 1)
    def _clipped():
        pltpu.sync_copy(acc_sh.at[pl.ds((NS - 1) * rows_per_sub, rows_last)],
                        part_hbm.at[cid, pl.ds((NS - 1) * rows_per_sub, rows_last)])


def _sc_edgemul_body(out_hbm, src2_hbm, dst2_hbm, emb_hbm, *sc, nch):
    sidx_v, didx_v = sc[0], sc[1]
    srows = sc[2:2 + NBUF]
    drows = sc[2 + NBUF:2 + 2 * NBUF]
    gsS = sc[2 + 2 * NBUF:2 + 3 * NBUF]
    gsD = sc[2 + 3 * NBUF:2 + 4 * NBUF]
    wsem = sc[2 + 4 * NBUF:2 + 5 * NBUF]
    cid = lax.axis_index("c")
    sid = lax.axis_index("s")
    wid = sid * NC + cid

    base = wid * nch
    pltpu.sync_copy(src2_hbm.at[pl.ds(base, nch)], sidx_v)
    pltpu.sync_copy(dst2_hbm.at[pl.ds(base, nch)], didx_v)

    def _wr_dst(j):
        return emb_hbm.at[pl.ds((base + j) * CHUNK, CHUNK)]

    def _start_gathers(j, b):
        pltpu.async_copy(out_hbm.at[sidx_v.at[j]], srows[b], gsS[b])
        pltpu.async_copy(out_hbm.at[didx_v.at[j]], drows[b], gsD[b])

    def _wait_writeout(j, b):
        pltpu.make_async_copy(srows[b], _wr_dst(j), wsem[b]).wait()

    def _process(j, b):
        cur_s, cur_d = srows[b], drows[b]
        pltpu.make_async_copy(out_hbm.at[sidx_v.at[j]], cur_s, gsS[b]).wait()
        pltpu.make_async_copy(out_hbm.at[didx_v.at[j]], cur_d, gsD[b]).wait()

        @pl.loop(0, CHUNK, unroll=8)
        def _mul(r):
            cur_s[r, :] = cur_s[r, :] * cur_d[r, :]

        pltpu.async_copy(cur_s, _wr_dst(j), wsem[b])

    for k in range(PF):
        _start_gathers(k, k)

    main_end = ((nch - PF - 1) // NBUF) * NBUF

    @pl.loop(0, main_end, step=NBUF)
    def _main(jj):
        for b in range(NBUF):
            j = jj + b
            _process(j, b)
            b2 = (b + PF) % NBUF

            @pl.when(j >= PF)
            def _w():
                _wait_writeout(j, b2)

            _start_gathers(j + PF, b2)

    for j in range(main_end, nch):
        b = j % NBUF
        _process(j, b)
        if j >= PF:
            _wait_writeout(j, (j + PF) % NBUF)
        if j + PF < nch:
            _start_gathers(j + PF, (j + PF) % NBUF)
    for j in range(max(0, nch - PF), nch):
        _wait_writeout(j, j % NBUF)


def kernel(x, edge_index, W0, b0, We1, be1, We2, be2, Wroot, bconv, W1, b1,
           W2, b2):
    n, d = x.shape
    e = edge_index.shape[1]
    h_dim = W0.shape[1]

    src = edge_index[0]
    dst = edge_index[1]

    e1 = jax.nn.relu(We1[0] + be1)
    w_e = (e1 @ We2 + be2).reshape(h_dim, h_dim)

    P = 128 // h_dim
    eyeP = jnp.eye(P, dtype=jnp.float32)
    W0big = jnp.kron(eyeP, W0)
    b0big = jnp.tile(b0, P).reshape(1, P * h_dim)
    Wrootbig = jnp.kron(eyeP, Wroot)
    Webig = jnp.kron(eyeP, w_e)
    bconvbig = jnp.tile(bconv, P).reshape(1, P * h_dim)
    W1big = jnp.kron(eyeP, W1)
    b1big = jnp.tile(b1, P).reshape(1, P * 8)
    W2big = jnp.kron(eyeP, W2)

    h_p = pl.pallas_call(
        _tc1_body,
        out_shape=jax.ShapeDtypeStruct((n // P, P * h_dim), jnp.float32),
    )(x.reshape(n // P, P * d), W0big, b0big)
    h = h_p.reshape(n, h_dim)

    chunks = -(-e // (CHUNK * NW)) * NW
    nch = chunks // NW
    pad_e = chunks * CHUNK - e
    src2 = jnp.concatenate(
        [src, jnp.zeros((pad_e,), jnp.int32)]).reshape(chunks, CHUNK)
    dstA = jnp.concatenate(
        [dst, jnp.full((pad_e,), n, jnp.int32)]).reshape(chunks, CHUNK)
    dstB = jnp.concatenate(
        [dst, jnp.zeros((pad_e,), jnp.int32)]).reshape(chunks, CHUNK)
    rows_per_sub = -(-n // (8 * NS)) * 8
    npad = rows_per_sub * NS
    zchunk = rows_per_sub // 4
    mesh = plsc.VectorSubcoreMesh(core_axis_name="c", subcore_axis_name="s",
                                  num_cores=NC, num_subcores=NS)
    rows_last = n - rows_per_sub * (NS - 1)
    seg = functools.partial(_sc_segsum_body, nch=nch,
                            rows_per_sub=rows_per_sub, rows_last=rows_last)
    partials = pl.kernel(
        seg,
        out_type=jax.ShapeDtypeStruct((NC, n, h_dim), jnp.float32),
        mesh=mesh,
        scratch_types=(
            [pltpu.VMEM((nch, CHUNK), jnp.int32)] * 2
            + [pltpu.VMEM((CHUNK, h_dim), jnp.float32)] * NBUF
            + [pltpu.VMEM((zchunk, h_dim), jnp.float32),
               pltpu.VMEM_SHARED((npad, h_dim), jnp.float32)]
            + [pltpu.SemaphoreType.DMA] * (2 * NBUF)
        ),
        compiler_params=pltpu.CompilerParams(use_tc_tiling_on_sc=False),
    )(h, src2, dstA)

    part_p = partials.reshape(NC, n // P, P * h_dim)
    out_p = pl.pallas_call(
        _tc2_body,
        out_shape=jax.ShapeDtypeStruct((n // P, P * h_dim), jnp.float32),
    )(h_p, part_p, Wrootbig, Webig, bconvbig)
    out = out_p.reshape(n, h_dim)

    mul = functools.partial(_sc_edgemul_body, nch=nch)
    emb = pl.kernel(
        mul,
        out_type=jax.ShapeDtypeStruct((chunks * CHUNK, h_dim), jnp.float32),
        mesh=mesh,
        scratch_types=(
            [pltpu.VMEM((nch, CHUNK), jnp.int32)] * 2
            + [pltpu.VMEM((CHUNK, h_dim), jnp.float32)] * (2 * NBUF)
            + [pltpu.SemaphoreType.DMA] * (3 * NBUF)
        ),
        compiler_params=pltpu.CompilerParams(use_tc_tiling_on_sc=False),
    )(out, src2, dstB)

    ep = chunks * CHUNK // P
    emb_p = emb.reshape(ep, P * h_dim)
    blk = ep // 4
    score = pl.pallas_call(
        _tc3_body,
        grid=(ep // blk,),
        in_specs=[
            pl.BlockSpec((blk, P * h_dim), lambda i: (i, 0)),
            pl.BlockSpec((P * h_dim, P * 8), lambda i: (0, 0)),
            pl.BlockSpec((1, P * 8), lambda i: (0, 0)),
            pl.BlockSpec((P * 8, P), lambda i: (0, 0)),
            pl.BlockSpec((1, 1), lambda i: (0, 0)),
        ],
        out_specs=pl.BlockSpec((P, blk), lambda i: (0, i)),
        out_shape=jax.ShapeDtypeStruct((P, ep), jnp.float32),
    )(emb_p, W1big, b1big, W2big, b2.reshape(1, 1))

    return score.T.reshape(-1)[:e]
